# Initial kernel scaffold; baseline (speedup 1.0000x reference)
import jax
import jax.numpy as jnp
from jax.experimental import pallas as pl


def kernel(input, coeffs_t, params):
    raise NotImplementedError("write your pallas kernel here")


# speedup vs baseline: 1.3937x; 1.3937x over previous
import jax
import jax.numpy as jnp
import numpy as np
from jax.experimental import pallas as pl
from jax.experimental.pallas import tpu as pltpu

E, K, NC, D, P, NP, H, DH = 8, 3, 100, 192, 4, 64, 3, 64
B = 1024
FLAT = 3 * 32 * 32
SEQ = NP + 1
SEQP = 72
CIN = 64
NCP = 128
BLK = 32
NBLK = B // BLK + E
SLOTS = NBLK * BLK
CHUNK = 65536

_HI = jax.lax.Precision.HIGHEST


def _blend_kern(c_ref, w_ref, o_ref):
    o_ref[...] = jnp.sum(w_ref[...] * c_ref[...], axis=0, keepdims=True)


def _route_kern(lg_ref, dest_ref, bexp_ref):
    logits = lg_ref[...]
    m = jnp.max(logits, axis=-1, keepdims=True)
    eiota = jax.lax.broadcasted_iota(jnp.int32, (B, E), 1)
    idx = jnp.min(jnp.where(logits == m, eiota, E), axis=-1, keepdims=True)
    oh = (idx == eiota).astype(jnp.float32)
    ii = jax.lax.broadcasted_iota(jnp.int32, (B, B), 0)
    jj = jax.lax.broadcasted_iota(jnp.int32, (B, B), 1)
    ltri = (jj < ii).astype(jnp.float32)
    rank_all = jnp.dot(ltri, oh, precision=_HI)
    rank = jnp.sum(rank_all * oh, axis=-1, keepdims=True)
    counts = jnp.sum(oh, axis=0, keepdims=True)
    padded = jnp.ceil(counts / BLK) * BLK
    er = jax.lax.broadcasted_iota(jnp.int32, (E, E), 0)
    ec = jax.lax.broadcasted_iota(jnp.int32, (E, E), 1)
    ustrict = (er < ec).astype(jnp.float32)
    offs = jnp.dot(padded, ustrict, precision=_HI)
    dest = jnp.sum(oh * offs, axis=-1, keepdims=True) + rank
    dest_ref[...] = dest.astype(jnp.int32)
    pstart = (jax.lax.broadcasted_iota(jnp.int32, (NBLK, E), 0) * BLK).astype(jnp.float32)
    within = (pstart >= offs) & (pstart < offs + padded)
    eid = jax.lax.broadcasted_iota(jnp.int32, (NBLK, E), 1)
    bexp_ref[...] = jnp.sum(jnp.where(within, eid, 0), axis=-1, keepdims=True)


def _gather_kern(dr_ref, xp_ref, xg_ref):
    b = pl.program_id(0)
    rows = jax.lax.broadcasted_iota(jnp.int32, (BLK, B), 0) + b * BLK
    oh = (rows == dr_ref[...]).astype(jnp.float32)
    xg_ref[...] = jnp.dot(oh, xp_ref[...], precision=_HI)


def _lnk(x, g, b):
    m = x.mean(-1, keepdims=True)
    v = ((x - m) ** 2).mean(-1, keepdims=True)
    return (x - m) / jnp.sqrt(v + 1e-6) * g + b


def _expert_kern(bexp, xg_ref, pw_ref, pos_ref, ln1_ref, wq_ref, wk_ref, wv_ref,
                 qb_ref, pj_ref, pjb_ref, ln2_ref, w1_ref, b1_ref, w2_ref, b2_ref,
                 lnf_ref, hw_ref, hb_ref, out_ref):
    xg = xg_ref[...]
    t2 = jnp.dot(xg, pw_ref[0], precision=_HI)
    t = t2.reshape(BLK, SEQP, D) + pos_ref[0]
    g1 = ln1_ref[0, 0:1, :]
    b1n = ln1_ref[0, 1:2, :]
    h = _lnk(t, g1, b1n)
    h2 = h.reshape(BLK * SEQP, D)
    colmask = (jax.lax.broadcasted_iota(jnp.int32, (1, 1, SEQP), 2) < SEQ)
    attn_out = jnp.zeros((BLK * SEQP, D), dtype=jnp.float32)
    for hh in range(H):
        q = jnp.dot(h2, wq_ref[0, hh], precision=_HI) + qb_ref[0, hh:hh + 1, :]
        k = jnp.dot(h2, wk_ref[0, hh], precision=_HI) + qb_ref[0, H + hh:H + hh + 1, :]
        v = jnp.dot(h2, wv_ref[0, hh], precision=_HI) + qb_ref[0, 2 * H + hh:2 * H + hh + 1, :]
        q3 = q.reshape(BLK, SEQP, DH)
        k3 = k.reshape(BLK, SEQP, DH)
        v3 = v.reshape(BLK, SEQP, DH)
        sc = jax.lax.dot_general(q3, k3, (((2,), (2,)), ((0,), (0,))),
                                 precision=_HI) * (1.0 / np.sqrt(DH))
        sc = jnp.where(colmask, sc, -1e9)
        att = jax.nn.softmax(sc, axis=-1)
        o3 = jax.lax.dot_general(att, v3, (((2,), (1,)), ((0,), (0,))),
                                 precision=_HI)
        attn_out = attn_out + jnp.dot(o3.reshape(BLK * SEQP, DH), pj_ref[0, hh],
                                      precision=_HI)
    t = t + attn_out.reshape(BLK, SEQP, D) + pjb_ref[0]
    h = _lnk(t, ln2_ref[0, 0:1, :], ln2_ref[0, 1:2, :])
    h2 = h.reshape(BLK * SEQP, D)
    mh = jax.nn.gelu(jnp.dot(h2, w1_ref[0], precision=_HI) + b1_ref[0])
    m2 = jnp.dot(mh, w2_ref[0], precision=_HI) + b2_ref[0]
    t = t + m2.reshape(BLK, SEQP, D)
    f = t[:, 0, :]
    f = _lnk(f, lnf_ref[0, 0:1, :], lnf_ref[0, 1:2, :])
    out_ref[...] = jnp.dot(f, hw_ref[0], precision=_HI) + hb_ref[0]


def _unscatter_kern(dest_ref, so_ref, out_ref):
    cols = jax.lax.broadcasted_iota(jnp.int32, (B, SLOTS), 1)
    oh = (cols == dest_ref[...]).astype(jnp.float32)
    out_ref[...] = jnp.dot(oh, so_ref[...], precision=_HI)


def kernel(input, coeffs_t, params):
    x = input.astype(jnp.float32)
    f32 = jnp.float32

    order = ['patch_w', 'patch_b', 'cls', 'pos', 'ln1_g', 'ln1_b', 'qkv_w', 'qkv_b',
             'proj_w', 'proj_b', 'ln2_g', 'ln2_b', 'mlp_w1', 'mlp_b1', 'mlp_w2',
             'mlp_b2', 'lnf_g', 'lnf_b', 'head_w', 'head_b']
    flats, shapes, offsets = [], {}, {}
    off = 0
    for name in order:
        v = params[name]
        rest = v.shape[2:]
        flats.append(jnp.moveaxis(v, 1, 0).reshape(K, -1))
        shapes[name] = (E,) + rest
        offsets[name] = off
        off += int(np.prod(shapes[name]))
    big = jnp.concatenate(flats, axis=1)
    total = big.shape[1]
    nch = -(-total // CHUNK)
    big = jnp.pad(big, ((0, 0), (0, nch * CHUNK - total)))
    blended = pl.pallas_call(
        _blend_kern,
        grid=(nch,),
        in_specs=[pl.BlockSpec((K, 1), lambda i: (0, 0)),
                  pl.BlockSpec((K, CHUNK), lambda i: (0, i))],
        out_specs=pl.BlockSpec((1, CHUNK), lambda i: (0, i)),
        out_shape=jax.ShapeDtypeStruct((1, nch * CHUNK), f32),
    )(coeffs_t.reshape(K, 1).astype(f32), big)[0]

    def bl(name):
        o = offsets[name]
        n = int(np.prod(shapes[name]))
        return jax.lax.slice(blended, (o,), (o + n,)).reshape(shapes[name])

    patch_w, patch_b, cls_t, pos = bl('patch_w'), bl('patch_b'), bl('cls'), bl('pos')
    qkv_w, qkv_b = bl('qkv_w'), bl('qkv_b')
    proj_w, proj_b = bl('proj_w'), bl('proj_b')
    w1, b1v, w2, b2v = bl('mlp_w1'), bl('mlp_b1'), bl('mlp_w2'), bl('mlp_b2')
    hw, hb = bl('head_w'), bl('head_b')
    ln1 = jnp.stack([bl('ln1_g'), bl('ln1_b')], axis=1)
    ln2 = jnp.stack([bl('ln2_g'), bl('ln2_b')], axis=1)
    lnf = jnp.stack([bl('lnf_g'), bl('lnf_b')], axis=1)

    pw_pad = jnp.concatenate(
        [patch_w, cls_t, patch_b[:, None, :],
         jnp.zeros((E, CIN - 50, D), f32)], axis=1)
    pos_p = jnp.pad(pos, ((0, 0), (0, SEQP - SEQ), (0, 0)))
    wq = qkv_w[:, :, 0 * D:1 * D].reshape(E, D, H, DH).transpose(0, 2, 1, 3)
    wk = qkv_w[:, :, 1 * D:2 * D].reshape(E, D, H, DH).transpose(0, 2, 1, 3)
    wv = qkv_w[:, :, 2 * D:3 * D].reshape(E, D, H, DH).transpose(0, 2, 1, 3)
    qbr = qkv_b.reshape(E, 3 * H, DH)
    pj = proj_w.reshape(E, H, DH, D)
    pjb = proj_b[:, None, :]
    b1r = b1v[:, None, :]
    b2r = b2v[:, None, :]
    hw_p = jnp.pad(hw, ((0, 0), (0, 0), (0, NCP - NC)))
    hb_p = jnp.pad(hb, ((0, 0), (0, NCP - NC)))[:, None, :]

    rw = jnp.tensordot(coeffs_t, params['router_w'], axes=(0, 0))
    rb = jnp.tensordot(coeffs_t, params['router_b'], axes=(0, 0))
    logits = x.reshape(B, FLAT) @ rw + rb
    dest, bexp = pl.pallas_call(
        _route_kern,
        in_specs=[pl.BlockSpec((B, E), lambda: (0, 0))],
        out_specs=[pl.BlockSpec((B, 1), lambda: (0, 0)),
                   pl.BlockSpec((NBLK, 1), lambda: (0, 0))],
        out_shape=[jax.ShapeDtypeStruct((B, 1), jnp.int32),
                   jax.ShapeDtypeStruct((NBLK, 1), jnp.int32)],
    )(logits)

    patches = x.reshape(B, 3, 8, P, 8, P).transpose(0, 2, 4, 1, 3, 5).reshape(B, NP, 48)
    prow = jnp.concatenate(
        [patches, jnp.zeros((B, NP, 1), f32), jnp.ones((B, NP, 1), f32),
         jnp.zeros((B, NP, CIN - 50), f32)], axis=-1)
    c0 = jnp.zeros((CIN,), f32).at[48].set(1.0)
    row0 = jnp.broadcast_to(c0, (B, 1, CIN))
    xp = jnp.concatenate([row0, prow, jnp.zeros((B, SEQP - SEQ, CIN), f32)],
                         axis=1).reshape(B, SEQP * CIN)
    xg = pl.pallas_call(
        _gather_kern,
        grid=(NBLK,),
        in_specs=[pl.BlockSpec((1, B), lambda i: (0, 0)),
                  pl.BlockSpec((B, SEQP * CIN), lambda i: (0, 0))],
        out_specs=pl.BlockSpec((BLK, SEQP * CIN), lambda i: (i, 0)),
        out_shape=jax.ShapeDtypeStruct((SLOTS, SEQP * CIN), f32),
    )(dest.reshape(1, B), xp).reshape(SLOTS * SEQP, CIN)

    grid_spec = pltpu.PrefetchScalarGridSpec(
        num_scalar_prefetch=1,
        grid=(NBLK,),
        in_specs=[
            pl.BlockSpec((BLK * SEQP, CIN), lambda i, be: (i, 0)),
            pl.BlockSpec((1, CIN, D), lambda i, be: (be[i], 0, 0)),
            pl.BlockSpec((1, SEQP, D), lambda i, be: (be[i], 0, 0)),
            pl.BlockSpec((1, 2, D), lambda i, be: (be[i], 0, 0)),
            pl.BlockSpec((1, H, D, DH), lambda i, be: (be[i], 0, 0, 0)),
            pl.BlockSpec((1, H, D, DH), lambda i, be: (be[i], 0, 0, 0)),
            pl.BlockSpec((1, H, D, DH), lambda i, be: (be[i], 0, 0, 0)),
            pl.BlockSpec((1, 3 * H, DH), lambda i, be: (be[i], 0, 0)),
            pl.BlockSpec((1, H, DH, D), lambda i, be: (be[i], 0, 0, 0)),
            pl.BlockSpec((1, 1, D), lambda i, be: (be[i], 0, 0)),
            pl.BlockSpec((1, 2, D), lambda i, be: (be[i], 0, 0)),
            pl.BlockSpec((1, D, 4 * D), lambda i, be: (be[i], 0, 0)),
            pl.BlockSpec((1, 1, 4 * D), lambda i, be: (be[i], 0, 0)),
            pl.BlockSpec((1, 4 * D, D), lambda i, be: (be[i], 0, 0)),
            pl.BlockSpec((1, 1, D), lambda i, be: (be[i], 0, 0)),
            pl.BlockSpec((1, 2, D), lambda i, be: (be[i], 0, 0)),
            pl.BlockSpec((1, D, NCP), lambda i, be: (be[i], 0, 0)),
            pl.BlockSpec((1, 1, NCP), lambda i, be: (be[i], 0, 0)),
        ],
        out_specs=pl.BlockSpec((BLK, NCP), lambda i, be: (i, 0)),
    )
    slot_out = pl.pallas_call(
        _expert_kern,
        grid_spec=grid_spec,
        out_shape=jax.ShapeDtypeStruct((SLOTS, NCP), f32),
    )(bexp.reshape(NBLK), xg, pw_pad, pos_p, ln1, wq, wk, wv, qbr, pj, pjb,
      ln2, w1, b1r, w2, b2r, lnf, hw_p, hb_p)

    out = pl.pallas_call(
        _unscatter_kern,
        in_specs=[pl.BlockSpec((B, 1), lambda: (0, 0)),
                  pl.BlockSpec((SLOTS, NCP), lambda: (0, 0))],
        out_specs=pl.BlockSpec((B, NCP), lambda: (0, 0)),
        out_shape=jax.ShapeDtypeStruct((B, NCP), f32),
    )(dest, slot_out)
    return out[:, :NC]


# speedup vs baseline: 4.3346x; 3.1101x over previous
import jax
import jax.numpy as jnp
import numpy as np
from jax.experimental import pallas as pl
from jax.experimental.pallas import tpu as pltpu

E, K, NC, D, P, NP, H, DH = 8, 3, 100, 192, 4, 64, 3, 64
B = 1024
FLAT = 3 * 32 * 32
SEQ = NP + 1
SEQP = 72
CIN = 64
NCP = 128
BLK = 32
NBLK = B // BLK + E
SLOTS = NBLK * BLK
CHUNK = 65536

_HI = jax.lax.Precision.HIGHEST
_H3 = jax.lax.Precision.HIGH
_DF = jax.lax.Precision.DEFAULT


def _blend_kern(c_ref, w_ref, o_ref):
    o_ref[...] = jnp.sum(w_ref[...] * c_ref[...], axis=0, keepdims=True)


def _route_kern(lg_ref, dest_ref, bexp_ref):
    logits = lg_ref[...]
    m = jnp.max(logits, axis=-1, keepdims=True)
    eiota = jax.lax.broadcasted_iota(jnp.int32, (B, E), 1)
    idx = jnp.min(jnp.where(logits == m, eiota, E), axis=-1, keepdims=True)
    oh = (idx == eiota).astype(jnp.float32)
    ii = jax.lax.broadcasted_iota(jnp.int32, (B, B), 0)
    jj = jax.lax.broadcasted_iota(jnp.int32, (B, B), 1)
    ltri = (jj < ii).astype(jnp.float32)
    rank_all = jnp.dot(ltri, oh, precision=_HI)
    rank = jnp.sum(rank_all * oh, axis=-1, keepdims=True)
    counts = jnp.sum(oh, axis=0, keepdims=True)
    padded = jnp.ceil(counts / BLK) * BLK
    er = jax.lax.broadcasted_iota(jnp.int32, (E, E), 0)
    ec = jax.lax.broadcasted_iota(jnp.int32, (E, E), 1)
    ustrict = (er < ec).astype(jnp.float32)
    offs = jnp.dot(padded, ustrict, precision=_HI)
    dest = jnp.sum(oh * offs, axis=-1, keepdims=True) + rank
    dest_ref[...] = dest.astype(jnp.int32)
    pstart = (jax.lax.broadcasted_iota(jnp.int32, (NBLK, E), 0) * BLK).astype(jnp.float32)
    within = (pstart >= offs) & (pstart < offs + padded)
    eid = jax.lax.broadcasted_iota(jnp.int32, (NBLK, E), 1)
    bexp_ref[...] = jnp.sum(jnp.where(within, eid, 0), axis=-1, keepdims=True)


def _gather_kern(dr_ref, xp_ref, xg_ref):
    b = pl.program_id(0)
    rows = jax.lax.broadcasted_iota(jnp.int32, (BLK, B), 0) + b * BLK
    oh = (rows == dr_ref[...]).astype(jnp.float32)
    xg_ref[...] = jnp.dot(oh, xp_ref[...], precision=_DF)


def _lnk(x, g, b):
    m = x.mean(-1, keepdims=True)
    v = ((x - m) ** 2).mean(-1, keepdims=True)
    return (x - m) / jnp.sqrt(v + 1e-6) * g + b


def _expert_kern(bexp, xg_ref, pw_ref, pos_ref, ln1_ref, wq_ref, wk_ref, wv_ref,
                 qb_ref, pj_ref, pjb_ref, ln2_ref, w1_ref, b1_ref, w2_ref, b2_ref,
                 lnf_ref, hw_ref, hb_ref, out_ref):
    xg = xg_ref[...]
    t2 = jnp.dot(xg, pw_ref[0], precision=_DF)
    t = t2.reshape(BLK, SEQP, D) + pos_ref[0]
    g1 = ln1_ref[0, 0:1, :]
    b1n = ln1_ref[0, 1:2, :]
    h = _lnk(t, g1, b1n)
    h2 = h.reshape(BLK * SEQP, D)
    colmask = (jax.lax.broadcasted_iota(jnp.int32, (1, 1, SEQP), 2) < SEQ)
    attn_out = jnp.zeros((BLK * SEQP, D), dtype=jnp.float32)
    for hh in range(H):
        q = jnp.dot(h2, wq_ref[0, hh], precision=_DF) + qb_ref[0, hh:hh + 1, :]
        k = jnp.dot(h2, wk_ref[0, hh], precision=_DF) + qb_ref[0, H + hh:H + hh + 1, :]
        v = jnp.dot(h2, wv_ref[0, hh], precision=_DF) + qb_ref[0, 2 * H + hh:2 * H + hh + 1, :]
        q3 = q.reshape(BLK, SEQP, DH)
        k3 = k.reshape(BLK, SEQP, DH)
        v3 = v.reshape(BLK, SEQP, DH)
        sc = jax.lax.dot_general(q3, k3, (((2,), (2,)), ((0,), (0,))),
                                 precision=_DF) * (1.0 / np.sqrt(DH))
        sc = jnp.where(colmask, sc, -1e9)
        att = jax.nn.softmax(sc, axis=-1)
        o3 = jax.lax.dot_general(att, v3, (((2,), (1,)), ((0,), (0,))),
                                 precision=_DF)
        attn_out = attn_out + jnp.dot(o3.reshape(BLK * SEQP, DH), pj_ref[0, hh],
                                      precision=_DF)
    t = t + attn_out.reshape(BLK, SEQP, D) + pjb_ref[0]
    h = _lnk(t, ln2_ref[0, 0:1, :], ln2_ref[0, 1:2, :])
    h2 = h.reshape(BLK * SEQP, D)
    mh = jax.nn.gelu(jnp.dot(h2, w1_ref[0], precision=_DF) + b1_ref[0])
    m2 = jnp.dot(mh, w2_ref[0], precision=_DF) + b2_ref[0]
    t = t + m2.reshape(BLK, SEQP, D)
    f = t[:, 0, :]
    f = _lnk(f, lnf_ref[0, 0:1, :], lnf_ref[0, 1:2, :])
    out_ref[...] = jnp.dot(f, hw_ref[0], precision=_DF) + hb_ref[0]


def _unscatter_kern(dest_ref, so_ref, out_ref):
    cols = jax.lax.broadcasted_iota(jnp.int32, (B, SLOTS), 1)
    oh = (cols == dest_ref[...]).astype(jnp.float32)
    out_ref[...] = jnp.dot(oh, so_ref[...], precision=_HI)


def kernel(input, coeffs_t, params):
    x = input.astype(jnp.float32)
    f32 = jnp.float32

    order = ['patch_w', 'patch_b', 'cls', 'pos', 'ln1_g', 'ln1_b', 'qkv_w', 'qkv_b',
             'proj_w', 'proj_b', 'ln2_g', 'ln2_b', 'mlp_w1', 'mlp_b1', 'mlp_w2',
             'mlp_b2', 'lnf_g', 'lnf_b', 'head_w', 'head_b']
    flats, shapes, offsets = [], {}, {}
    off = 0
    for name in order:
        v = params[name]
        rest = v.shape[2:]
        flats.append(jnp.moveaxis(v, 1, 0).reshape(K, -1))
        shapes[name] = (E,) + rest
        offsets[name] = off
        off += int(np.prod(shapes[name]))
    big = jnp.concatenate(flats, axis=1)
    total = big.shape[1]
    nch = -(-total // CHUNK)
    big = jnp.pad(big, ((0, 0), (0, nch * CHUNK - total)))
    blended = pl.pallas_call(
        _blend_kern,
        grid=(nch,),
        in_specs=[pl.BlockSpec((K, 1), lambda i: (0, 0)),
                  pl.BlockSpec((K, CHUNK), lambda i: (0, i))],
        out_specs=pl.BlockSpec((1, CHUNK), lambda i: (0, i)),
        out_shape=jax.ShapeDtypeStruct((1, nch * CHUNK), f32),
    )(coeffs_t.reshape(K, 1).astype(f32), big)[0]

    def bl(name):
        o = offsets[name]
        n = int(np.prod(shapes[name]))
        return jax.lax.slice(blended, (o,), (o + n,)).reshape(shapes[name])

    patch_w, patch_b, cls_t, pos = bl('patch_w'), bl('patch_b'), bl('cls'), bl('pos')
    qkv_w, qkv_b = bl('qkv_w'), bl('qkv_b')
    proj_w, proj_b = bl('proj_w'), bl('proj_b')
    w1, b1v, w2, b2v = bl('mlp_w1'), bl('mlp_b1'), bl('mlp_w2'), bl('mlp_b2')
    hw, hb = bl('head_w'), bl('head_b')
    ln1 = jnp.stack([bl('ln1_g'), bl('ln1_b')], axis=1)
    ln2 = jnp.stack([bl('ln2_g'), bl('ln2_b')], axis=1)
    lnf = jnp.stack([bl('lnf_g'), bl('lnf_b')], axis=1)

    pw_pad = jnp.concatenate(
        [patch_w, cls_t, patch_b[:, None, :],
         jnp.zeros((E, CIN - 50, D), f32)], axis=1)
    pos_p = jnp.pad(pos, ((0, 0), (0, SEQP - SEQ), (0, 0)))
    wq = qkv_w[:, :, 0 * D:1 * D].reshape(E, D, H, DH).transpose(0, 2, 1, 3)
    wk = qkv_w[:, :, 1 * D:2 * D].reshape(E, D, H, DH).transpose(0, 2, 1, 3)
    wv = qkv_w[:, :, 2 * D:3 * D].reshape(E, D, H, DH).transpose(0, 2, 1, 3)
    qbr = qkv_b.reshape(E, 3 * H, DH)
    pj = proj_w.reshape(E, H, DH, D)
    pjb = proj_b[:, None, :]
    b1r = b1v[:, None, :]
    b2r = b2v[:, None, :]
    hw_p = jnp.pad(hw, ((0, 0), (0, 0), (0, NCP - NC)))
    hb_p = jnp.pad(hb, ((0, 0), (0, NCP - NC)))[:, None, :]

    rw = jnp.tensordot(coeffs_t, params['router_w'], axes=(0, 0))
    rb = jnp.tensordot(coeffs_t, params['router_b'], axes=(0, 0))
    logits = x.reshape(B, FLAT) @ rw + rb
    dest, bexp = pl.pallas_call(
        _route_kern,
        in_specs=[pl.BlockSpec((B, E), lambda: (0, 0))],
        out_specs=[pl.BlockSpec((B, 1), lambda: (0, 0)),
                   pl.BlockSpec((NBLK, 1), lambda: (0, 0))],
        out_shape=[jax.ShapeDtypeStruct((B, 1), jnp.int32),
                   jax.ShapeDtypeStruct((NBLK, 1), jnp.int32)],
    )(logits)

    patches = x.reshape(B, 3, 8, P, 8, P).transpose(0, 2, 4, 1, 3, 5).reshape(B, NP, 48)
    prow = jnp.concatenate(
        [patches, jnp.zeros((B, NP, 1), f32), jnp.ones((B, NP, 1), f32),
         jnp.zeros((B, NP, CIN - 50), f32)], axis=-1)
    c0 = jnp.zeros((CIN,), f32).at[48].set(1.0)
    row0 = jnp.broadcast_to(c0, (B, 1, CIN))
    xp = jnp.concatenate([row0, prow, jnp.zeros((B, SEQP - SEQ, CIN), f32)],
                         axis=1).reshape(B, SEQP * CIN)
    xg = pl.pallas_call(
        _gather_kern,
        grid=(NBLK,),
        in_specs=[pl.BlockSpec((1, B), lambda i: (0, 0)),
                  pl.BlockSpec((B, SEQP * CIN), lambda i: (0, 0))],
        out_specs=pl.BlockSpec((BLK, SEQP * CIN), lambda i: (i, 0)),
        out_shape=jax.ShapeDtypeStruct((SLOTS, SEQP * CIN), f32),
    )(dest.reshape(1, B), xp).reshape(SLOTS * SEQP, CIN)

    grid_spec = pltpu.PrefetchScalarGridSpec(
        num_scalar_prefetch=1,
        grid=(NBLK,),
        in_specs=[
            pl.BlockSpec((BLK * SEQP, CIN), lambda i, be: (i, 0)),
            pl.BlockSpec((1, CIN, D), lambda i, be: (be[i], 0, 0)),
            pl.BlockSpec((1, SEQP, D), lambda i, be: (be[i], 0, 0)),
            pl.BlockSpec((1, 2, D), lambda i, be: (be[i], 0, 0)),
            pl.BlockSpec((1, H, D, DH), lambda i, be: (be[i], 0, 0, 0)),
            pl.BlockSpec((1, H, D, DH), lambda i, be: (be[i], 0, 0, 0)),
            pl.BlockSpec((1, H, D, DH), lambda i, be: (be[i], 0, 0, 0)),
            pl.BlockSpec((1, 3 * H, DH), lambda i, be: (be[i], 0, 0)),
            pl.BlockSpec((1, H, DH, D), lambda i, be: (be[i], 0, 0, 0)),
            pl.BlockSpec((1, 1, D), lambda i, be: (be[i], 0, 0)),
            pl.BlockSpec((1, 2, D), lambda i, be: (be[i], 0, 0)),
            pl.BlockSpec((1, D, 4 * D), lambda i, be: (be[i], 0, 0)),
            pl.BlockSpec((1, 1, 4 * D), lambda i, be: (be[i], 0, 0)),
            pl.BlockSpec((1, 4 * D, D), lambda i, be: (be[i], 0, 0)),
            pl.BlockSpec((1, 1, D), lambda i, be: (be[i], 0, 0)),
            pl.BlockSpec((1, 2, D), lambda i, be: (be[i], 0, 0)),
            pl.BlockSpec((1, D, NCP), lambda i, be: (be[i], 0, 0)),
            pl.BlockSpec((1, 1, NCP), lambda i, be: (be[i], 0, 0)),
        ],
        out_specs=pl.BlockSpec((BLK, NCP), lambda i, be: (i, 0)),
    )
    slot_out = pl.pallas_call(
        _expert_kern,
        grid_spec=grid_spec,
        out_shape=jax.ShapeDtypeStruct((SLOTS, NCP), f32),
    )(bexp.reshape(NBLK), xg, pw_pad, pos_p, ln1, wq, wk, wv, qbr, pj, pjb,
      ln2, w1, b1r, w2, b2r, lnf, hw_p, hb_p)

    out = pl.pallas_call(
        _unscatter_kern,
        in_specs=[pl.BlockSpec((B, 1), lambda: (0, 0)),
                  pl.BlockSpec((SLOTS, NCP), lambda: (0, 0))],
        out_specs=pl.BlockSpec((B, NCP), lambda: (0, 0)),
        out_shape=jax.ShapeDtypeStruct((B, NCP), f32),
    )(dest, slot_out)
    return out[:, :NC]


# speedup vs baseline: 6.1304x; 1.4143x over previous
import jax
import jax.numpy as jnp
import numpy as np
from jax.experimental import pallas as pl
from jax.experimental.pallas import tpu as pltpu

E, K, NC, D, P, NP, H, DH = 8, 3, 100, 192, 4, 64, 3, 64
B = 1024
FLAT = 3 * 32 * 32
SEQ = NP + 1
SEQP = 72
CIN = 64
NCP = 128
BLK = 32
NBLK = B // BLK + E
SLOTS = NBLK * BLK

_HI = jax.lax.Precision.HIGHEST
_DF = jax.lax.Precision.DEFAULT

_BLEND_NAMES = ['patch_w', 'patch_b', 'cls', 'pos', 'ln1_g', 'ln1_b', 'qkv_w',
                'qkv_b', 'proj_w', 'proj_b', 'ln2_g', 'ln2_b', 'mlp_w1',
                'mlp_b1', 'mlp_w2', 'mlp_b2', 'lnf_g', 'lnf_b', 'head_w',
                'head_b']


def _blend_kern(s_ref, *refs):
    n = len(refs) // 2
    s = s_ref[...]
    for i_ref, o_ref in zip(refs[:n], refs[n:]):
        o_ref[...] = jnp.dot(s, i_ref[...], precision=_DF)


def _route_kern(lg_ref, dest_ref, bexp_ref):
    logits = lg_ref[...]
    m = jnp.max(logits, axis=-1, keepdims=True)
    eiota = jax.lax.broadcasted_iota(jnp.int32, (B, E), 1)
    idx = jnp.min(jnp.where(logits == m, eiota, E), axis=-1, keepdims=True)
    oh = (idx == eiota).astype(jnp.float32)
    ii = jax.lax.broadcasted_iota(jnp.int32, (B, B), 0)
    jj = jax.lax.broadcasted_iota(jnp.int32, (B, B), 1)
    ltri = (jj < ii).astype(jnp.float32)
    rank_all = jnp.dot(ltri, oh, precision=_HI)
    rank = jnp.sum(rank_all * oh, axis=-1, keepdims=True)
    counts = jnp.sum(oh, axis=0, keepdims=True)
    padded = jnp.ceil(counts / BLK) * BLK
    er = jax.lax.broadcasted_iota(jnp.int32, (E, E), 0)
    ec = jax.lax.broadcasted_iota(jnp.int32, (E, E), 1)
    ustrict = (er < ec).astype(jnp.float32)
    offs = jnp.dot(padded, ustrict, precision=_HI)
    dest = jnp.sum(oh * offs, axis=-1, keepdims=True) + rank
    dest_ref[...] = dest.astype(jnp.int32)
    pstart = (jax.lax.broadcasted_iota(jnp.int32, (NBLK, E), 0) * BLK).astype(jnp.float32)
    within = (pstart >= offs) & (pstart < offs + padded)
    eid = jax.lax.broadcasted_iota(jnp.int32, (NBLK, E), 1)
    bexp_ref[...] = jnp.sum(jnp.where(within, eid, 0), axis=-1, keepdims=True)


def _gather_kern(dr_ref, xp_ref, xg_ref):
    b = pl.program_id(0)
    rows = jax.lax.broadcasted_iota(jnp.int32, (BLK, B), 0) + b * BLK
    oh = (rows == dr_ref[...]).astype(jnp.float32)
    xg_ref[...] = jnp.dot(oh, xp_ref[...], precision=_DF)


def _lnk(x, g, b):
    m = x.mean(-1, keepdims=True)
    v = ((x - m) ** 2).mean(-1, keepdims=True)
    return (x - m) / jnp.sqrt(v + 1e-6) * g + b


def _expert_kern(bexp, xg_ref, pw_ref, pos_ref, ln1_ref, qkvw_ref, qkvb_ref,
                 pj_ref, pjb_ref, ln2_ref, w1_ref, b1_ref, w2_ref, b2_ref,
                 lnf_ref, hw_ref, hb_ref, out_ref):
    xg = xg_ref[...]
    t2 = jnp.dot(xg, pw_ref[0], precision=_DF)
    t = t2.reshape(BLK, SEQP, D) + pos_ref[0]
    h = _lnk(t, ln1_ref[0, 0:1, :], ln1_ref[0, 1:2, :])
    h2 = h.reshape(BLK * SEQP, D)
    qkv = jnp.dot(h2, qkvw_ref[0], precision=_DF) + qkvb_ref[0]
    colmask = (jax.lax.broadcasted_iota(jnp.int32, (1, 1, SEQP), 2) < SEQ)
    attn_out = jnp.zeros((BLK * SEQP, D), dtype=jnp.float32)
    for hh in range(H):
        q3 = qkv[:, hh * DH:(hh + 1) * DH].reshape(BLK, SEQP, DH)
        k3 = qkv[:, D + hh * DH:D + (hh + 1) * DH].reshape(BLK, SEQP, DH)
        v3 = qkv[:, 2 * D + hh * DH:2 * D + (hh + 1) * DH].reshape(BLK, SEQP, DH)
        sc = jax.lax.dot_general(q3, k3, (((2,), (2,)), ((0,), (0,))),
                                 precision=_DF) * (1.0 / np.sqrt(DH))
        sc = jnp.where(colmask, sc, -1e9)
        att = jax.nn.softmax(sc, axis=-1)
        o3 = jax.lax.dot_general(att, v3, (((2,), (1,)), ((0,), (0,))),
                                 precision=_DF)
        attn_out = attn_out + jnp.dot(o3.reshape(BLK * SEQP, DH), pj_ref[0, hh],
                                      precision=_DF)
    t = t + attn_out.reshape(BLK, SEQP, D) + pjb_ref[0]
    h = _lnk(t, ln2_ref[0, 0:1, :], ln2_ref[0, 1:2, :])
    h2 = h.reshape(BLK * SEQP, D)
    mh = jax.nn.gelu(jnp.dot(h2, w1_ref[0], precision=_DF) + b1_ref[0])
    m2 = jnp.dot(mh, w2_ref[0], precision=_DF) + b2_ref[0]
    t = t + m2.reshape(BLK, SEQP, D)
    f = t[:, 0, :]
    f = _lnk(f, lnf_ref[0, 0:1, :], lnf_ref[0, 1:2, :])
    out_ref[...] = jnp.dot(f, hw_ref[0], precision=_DF) + hb_ref[0]


def _unscatter_kern(dest_ref, so_ref, out_ref):
    cols = jax.lax.broadcasted_iota(jnp.int32, (B, SLOTS), 1)
    oh = (cols == dest_ref[...]).astype(jnp.float32)
    out_ref[...] = jnp.dot(oh, so_ref[...], precision=_HI)


def kernel(input, coeffs_t, params):
    x = input.astype(jnp.float32)
    f32 = jnp.float32

    ins, shapes = [], {}
    for name in _BLEND_NAMES:
        v = params[name]
        shapes[name] = (E,) + v.shape[2:]
        ins.append(v.reshape(E * K, -1))
    sel = jnp.kron(jnp.eye(E, dtype=f32), coeffs_t.astype(f32)[None, :])
    outs = []
    for lo, hi in ((0, 13), (13, len(ins))):
        grp = ins[lo:hi]
        outs += pl.pallas_call(
            _blend_kern,
            in_specs=([pl.BlockSpec((E, E * K), lambda: (0, 0))] +
                      [pl.BlockSpec(a.shape, lambda: (0, 0)) for a in grp]),
            out_specs=[pl.BlockSpec((E, a.shape[1]), lambda: (0, 0)) for a in grp],
            out_shape=[jax.ShapeDtypeStruct((E, a.shape[1]), f32) for a in grp],
        )(sel, *grp)
    bld = {name: o.reshape(shapes[name]) for name, o in zip(_BLEND_NAMES, outs)}

    pw_pad = jnp.concatenate(
        [bld['patch_w'], bld['cls'], bld['patch_b'][:, None, :],
         jnp.zeros((E, CIN - 50, D), f32)], axis=1)
    pos_p = jnp.pad(bld['pos'], ((0, 0), (0, SEQP - SEQ), (0, 0)))
    qkvw = bld['qkv_w']
    qkvb = bld['qkv_b'][:, None, :]
    pj = bld['proj_w'].reshape(E, H, DH, D)
    pjb = bld['proj_b'][:, None, :]
    w1, b1r = bld['mlp_w1'], bld['mlp_b1'][:, None, :]
    w2, b2r = bld['mlp_w2'], bld['mlp_b2'][:, None, :]
    hw_p = jnp.pad(bld['head_w'], ((0, 0), (0, 0), (0, NCP - NC)))
    hb_p = jnp.pad(bld['head_b'], ((0, 0), (0, NCP - NC)))[:, None, :]
    ln1 = jnp.stack([bld['ln1_g'], bld['ln1_b']], axis=1)
    ln2 = jnp.stack([bld['ln2_g'], bld['ln2_b']], axis=1)
    lnf = jnp.stack([bld['lnf_g'], bld['lnf_b']], axis=1)

    rw = jnp.tensordot(coeffs_t, params['router_w'], axes=(0, 0))
    rb = jnp.tensordot(coeffs_t, params['router_b'], axes=(0, 0))
    logits = x.reshape(B, FLAT) @ rw + rb
    dest, bexp = pl.pallas_call(
        _route_kern,
        in_specs=[pl.BlockSpec((B, E), lambda: (0, 0))],
        out_specs=[pl.BlockSpec((B, 1), lambda: (0, 0)),
                   pl.BlockSpec((NBLK, 1), lambda: (0, 0))],
        out_shape=[jax.ShapeDtypeStruct((B, 1), jnp.int32),
                   jax.ShapeDtypeStruct((NBLK, 1), jnp.int32)],
    )(logits)

    patches = x.reshape(B, 3, 8, P, 8, P).transpose(0, 2, 4, 1, 3, 5).reshape(B, NP, 48)
    prow = jnp.concatenate(
        [patches, jnp.zeros((B, NP, 1), f32), jnp.ones((B, NP, 1), f32),
         jnp.zeros((B, NP, CIN - 50), f32)], axis=-1)
    c0 = jnp.zeros((CIN,), f32).at[48].set(1.0)
    row0 = jnp.broadcast_to(c0, (B, 1, CIN))
    xp = jnp.concatenate([row0, prow, jnp.zeros((B, SEQP - SEQ, CIN), f32)],
                         axis=1).reshape(B, SEQP * CIN)
    xg = pl.pallas_call(
        _gather_kern,
        grid=(NBLK,),
        in_specs=[pl.BlockSpec((1, B), lambda i: (0, 0)),
                  pl.BlockSpec((B, SEQP * CIN), lambda i: (0, 0))],
        out_specs=pl.BlockSpec((BLK, SEQP * CIN), lambda i: (i, 0)),
        out_shape=jax.ShapeDtypeStruct((SLOTS, SEQP * CIN), f32),
    )(dest.reshape(1, B), xp).reshape(SLOTS * SEQP, CIN)

    grid_spec = pltpu.PrefetchScalarGridSpec(
        num_scalar_prefetch=1,
        grid=(NBLK,),
        in_specs=[
            pl.BlockSpec((BLK * SEQP, CIN), lambda i, be: (i, 0)),
            pl.BlockSpec((1, CIN, D), lambda i, be: (be[i], 0, 0)),
            pl.BlockSpec((1, SEQP, D), lambda i, be: (be[i], 0, 0)),
            pl.BlockSpec((1, 2, D), lambda i, be: (be[i], 0, 0)),
            pl.BlockSpec((1, D, 3 * D), lambda i, be: (be[i], 0, 0)),
            pl.BlockSpec((1, 1, 3 * D), lambda i, be: (be[i], 0, 0)),
            pl.BlockSpec((1, H, DH, D), lambda i, be: (be[i], 0, 0, 0)),
            pl.BlockSpec((1, 1, D), lambda i, be: (be[i], 0, 0)),
            pl.BlockSpec((1, 2, D), lambda i, be: (be[i], 0, 0)),
            pl.BlockSpec((1, D, 4 * D), lambda i, be: (be[i], 0, 0)),
            pl.BlockSpec((1, 1, 4 * D), lambda i, be: (be[i], 0, 0)),
            pl.BlockSpec((1, 4 * D, D), lambda i, be: (be[i], 0, 0)),
            pl.BlockSpec((1, 1, D), lambda i, be: (be[i], 0, 0)),
            pl.BlockSpec((1, 2, D), lambda i, be: (be[i], 0, 0)),
            pl.BlockSpec((1, D, NCP), lambda i, be: (be[i], 0, 0)),
            pl.BlockSpec((1, 1, NCP), lambda i, be: (be[i], 0, 0)),
        ],
        out_specs=pl.BlockSpec((BLK, NCP), lambda i, be: (i, 0)),
    )
    slot_out = pl.pallas_call(
        _expert_kern,
        grid_spec=grid_spec,
        out_shape=jax.ShapeDtypeStruct((SLOTS, NCP), f32),
    )(bexp.reshape(NBLK), xg, pw_pad, pos_p, ln1, qkvw, qkvb, pj, pjb,
      ln2, w1, b1r, w2, b2r, lnf, hw_p, hb_p)

    out = pl.pallas_call(
        _unscatter_kern,
        in_specs=[pl.BlockSpec((B, 1), lambda: (0, 0)),
                  pl.BlockSpec((SLOTS, NCP), lambda: (0, 0))],
        out_specs=pl.BlockSpec((B, NCP), lambda: (0, 0)),
        out_shape=jax.ShapeDtypeStruct((B, NCP), f32),
    )(dest, slot_out)
    return out[:, :NC]


# speedup vs baseline: 6.1469x; 1.0027x over previous
import jax
import jax.numpy as jnp
import numpy as np
from jax.experimental import pallas as pl
from jax.experimental.pallas import tpu as pltpu
from jax.experimental.pallas import tpu_sc as plsc

E, K, NC, D, P, NP, H, DH = 8, 3, 100, 192, 4, 64, 3, 64
B = 1024
FLAT = 3 * 32 * 32
SEQ = NP + 1
SEQP = 72
CIN = 64
NCP = 128
BLK = 32
NBLK = B // BLK + E
SLOTS = NBLK * BLK
DW = SEQP * CIN
SC_NC, SC_NS = 2, 16
NW = SC_NC * SC_NS
GPW = SLOTS // NW
GCH = 8
UPW = B // NW

_HI = jax.lax.Precision.HIGHEST
_DF = jax.lax.Precision.DEFAULT

_BLEND_NAMES = ['patch_w', 'patch_b', 'cls', 'pos', 'ln1_g', 'ln1_b', 'qkv_w',
                'qkv_b', 'proj_w', 'proj_b', 'ln2_g', 'ln2_b', 'mlp_w1',
                'mlp_b1', 'mlp_w2', 'mlp_b2', 'lnf_g', 'lnf_b', 'head_w',
                'head_b']


def _blend_kern(s_ref, *refs):
    n = len(refs) // 2
    s = s_ref[...]
    for i_ref, o_ref in zip(refs[:n], refs[n:]):
        o_ref[...] = jnp.dot(s, i_ref[...], precision=_DF)


def _route_kern(lg_ref, dest_ref, bexp_ref):
    logits = lg_ref[...]
    m = jnp.max(logits, axis=-1, keepdims=True)
    eiota = jax.lax.broadcasted_iota(jnp.int32, (B, E), 1)
    idx = jnp.min(jnp.where(logits == m, eiota, E), axis=-1, keepdims=True)
    oh = (idx == eiota).astype(jnp.float32)
    ii = jax.lax.broadcasted_iota(jnp.int32, (B, B), 0)
    jj = jax.lax.broadcasted_iota(jnp.int32, (B, B), 1)
    ltri = (jj < ii).astype(jnp.float32)
    rank_all = jnp.dot(ltri, oh, precision=_HI)
    rank = jnp.sum(rank_all * oh, axis=-1, keepdims=True)
    counts = jnp.sum(oh, axis=0, keepdims=True)
    padded = jnp.ceil(counts / BLK) * BLK
    er = jax.lax.broadcasted_iota(jnp.int32, (E, E), 0)
    ec = jax.lax.broadcasted_iota(jnp.int32, (E, E), 1)
    ustrict = (er < ec).astype(jnp.float32)
    offs = jnp.dot(padded, ustrict, precision=_HI)
    dest = jnp.sum(oh * offs, axis=-1, keepdims=True) + rank
    dest_ref[...] = dest.astype(jnp.int32)
    rowi = jax.lax.broadcasted_iota(jnp.int32, (NBLK + 1, E), 0)
    pstart = (rowi * BLK).astype(jnp.float32)
    within = (pstart >= offs) & (pstart < offs + padded)
    eid = jax.lax.broadcasted_iota(jnp.int32, (NBLK + 1, E), 1)
    base = jnp.sum(jnp.where(within, eid, 0), axis=-1, keepdims=True)
    nused = (jnp.sum(padded) * (1.0 / BLK)).astype(jnp.int32)
    bexp_ref[...] = base + jnp.where(rowi[:, :1] == NBLK, nused, 0)


def _gather_kern(dr_ref, xp_ref, xg_ref):
    b = pl.program_id(0)
    rows = jax.lax.broadcasted_iota(jnp.int32, (BLK, B), 0) + b * BLK
    oh = (rows == dr_ref[...]).astype(jnp.float32)
    xg_ref[...] = jnp.dot(oh, xp_ref[...], precision=_DF)


def _invert_kern(dr_ref, src_ref):
    rows = jax.lax.broadcasted_iota(jnp.int32, (SLOTS, B), 0)
    ohs = (rows == dr_ref[...]).astype(jnp.float32)
    ic = jax.lax.broadcasted_iota(jnp.int32, (B, 1), 0).astype(jnp.float32)
    valid = jnp.sum(ohs, axis=-1, keepdims=True)
    srcf = jnp.dot(ohs, ic, precision=_HI) + (1.0 - valid) * B
    src_ref[...] = srcf.astype(jnp.int32)


def _sc_gather_body(tab_ref, idx_ref, out_ref, idx_v, rows_v, sem):
    wid = jax.lax.axis_index("s") * SC_NC + jax.lax.axis_index("c")
    base = wid * GPW
    for c in range(GPW // GCH):
        off = base + c * GCH
        pltpu.sync_copy(idx_ref.at[pl.ds(off, GCH)], idx_v)
        pltpu.async_copy(tab_ref.at[idx_v], rows_v, sem).wait()
        pltpu.sync_copy(rows_v, out_ref.at[pl.ds(off, GCH)])


def _sc_unscatter_body(tab_ref, idx_ref, out_ref, idx_v, rows_v, sem):
    wid = jax.lax.axis_index("s") * SC_NC + jax.lax.axis_index("c")
    base = wid * UPW
    pltpu.sync_copy(idx_ref.at[pl.ds(base, UPW)], idx_v)
    pltpu.async_copy(tab_ref.at[idx_v], rows_v, sem).wait()
    pltpu.sync_copy(rows_v, out_ref.at[pl.ds(base, UPW)])


def _lnk(x, g, b):
    m = x.mean(-1, keepdims=True)
    v = ((x - m) ** 2).mean(-1, keepdims=True)
    return (x - m) / jnp.sqrt(v + 1e-6) * g + b


def _expert_kern(bexp, xg_ref, pw_ref, pos_ref, ln1_ref, qkvw_ref, qkvb_ref,
                 pj_ref, pjb_ref, ln2_ref, w1_ref, b1_ref, w2_ref, b2_ref,
                 lnf_ref, hw_ref, hb_ref, out_ref):
    nused = bexp[NBLK]

    @pl.when(pl.program_id(0) < nused)
    def _body():
        _expert_compute(bexp, xg_ref, pw_ref, pos_ref, ln1_ref, qkvw_ref,
                        qkvb_ref, pj_ref, pjb_ref, ln2_ref, w1_ref, b1_ref,
                        w2_ref, b2_ref, lnf_ref, hw_ref, hb_ref, out_ref)


def _expert_compute(bexp, xg_ref, pw_ref, pos_ref, ln1_ref, qkvw_ref, qkvb_ref,
                    pj_ref, pjb_ref, ln2_ref, w1_ref, b1_ref, w2_ref, b2_ref,
                    lnf_ref, hw_ref, hb_ref, out_ref):
    xg = xg_ref[...]
    t2 = jnp.dot(xg, pw_ref[0], precision=_DF)
    t = t2.reshape(BLK, SEQP, D) + pos_ref[0]
    h = _lnk(t, ln1_ref[0, 0:1, :], ln1_ref[0, 1:2, :])
    h2 = h.reshape(BLK * SEQP, D)
    qkv = jnp.dot(h2, qkvw_ref[0], precision=_DF) + qkvb_ref[0]
    colmask = (jax.lax.broadcasted_iota(jnp.int32, (1, 1, SEQP), 2) < SEQ)
    o_parts = []
    for hh in range(H):
        q3 = qkv[:, hh * DH:(hh + 1) * DH].reshape(BLK, SEQP, DH)
        k3 = qkv[:, D + hh * DH:D + (hh + 1) * DH].reshape(BLK, SEQP, DH)
        v3 = qkv[:, 2 * D + hh * DH:2 * D + (hh + 1) * DH].reshape(BLK, SEQP, DH)
        sc = jax.lax.dot_general(q3, k3, (((2,), (2,)), ((0,), (0,))),
                                 precision=_DF) * (1.0 / np.sqrt(DH))
        sc = jnp.where(colmask, sc, -1e9)
        att = jax.nn.softmax(sc, axis=-1)
        o3 = jax.lax.dot_general(att, v3, (((2,), (1,)), ((0,), (0,))),
                                 precision=_DF)
        o_parts.append(o3.reshape(BLK * SEQP, DH))
    o_all = jnp.concatenate(o_parts, axis=-1)
    attn_out = jnp.dot(o_all, pj_ref[0], precision=_DF)
    t = t + attn_out.reshape(BLK, SEQP, D) + pjb_ref[0]
    h = _lnk(t, ln2_ref[0, 0:1, :], ln2_ref[0, 1:2, :])
    h2 = h.reshape(BLK * SEQP, D)
    mh = jax.nn.gelu(jnp.dot(h2, w1_ref[0], precision=_DF) + b1_ref[0])
    m2 = jnp.dot(mh, w2_ref[0], precision=_DF) + b2_ref[0]
    t = t + m2.reshape(BLK, SEQP, D)
    f = t[:, 0, :]
    f = _lnk(f, lnf_ref[0, 0:1, :], lnf_ref[0, 1:2, :])
    out_ref[...] = jnp.dot(f, hw_ref[0], precision=_DF) + hb_ref[0]


def _unscatter_kern(dest_ref, so_ref, out_ref):
    cols = jax.lax.broadcasted_iota(jnp.int32, (B, SLOTS), 1)
    oh = (cols == dest_ref[...]).astype(jnp.float32)
    out_ref[...] = jnp.dot(oh, so_ref[...], precision=_HI)


def kernel(input, coeffs_t, params):
    x = input.astype(jnp.float32)
    f32 = jnp.float32

    ins, shapes = [], {}
    for name in _BLEND_NAMES:
        v = params[name]
        shapes[name] = (E,) + v.shape[2:]
        ins.append(v.reshape(E * K, -1))
    sel = jnp.kron(jnp.eye(E, dtype=f32), coeffs_t.astype(f32)[None, :])
    outs = []
    for lo, hi in ((0, 13), (13, len(ins))):
        grp = ins[lo:hi]
        outs += pl.pallas_call(
            _blend_kern,
            in_specs=([pl.BlockSpec((E, E * K), lambda: (0, 0))] +
                      [pl.BlockSpec(a.shape, lambda: (0, 0)) for a in grp]),
            out_specs=[pl.BlockSpec((E, a.shape[1]), lambda: (0, 0)) for a in grp],
            out_shape=[jax.ShapeDtypeStruct((E, a.shape[1]), f32) for a in grp],
        )(sel, *grp)
    bld = {name: o.reshape(shapes[name]) for name, o in zip(_BLEND_NAMES, outs)}

    pw_pad = jnp.concatenate(
        [bld['patch_w'], bld['cls'], bld['patch_b'][:, None, :],
         jnp.zeros((E, CIN - 50, D), f32)], axis=1)
    pos_p = jnp.pad(bld['pos'], ((0, 0), (0, SEQP - SEQ), (0, 0)))
    qkvw = bld['qkv_w']
    qkvb = bld['qkv_b'][:, None, :]
    pj = bld['proj_w']
    pjb = bld['proj_b'][:, None, :]
    w1, b1r = bld['mlp_w1'], bld['mlp_b1'][:, None, :]
    w2, b2r = bld['mlp_w2'], bld['mlp_b2'][:, None, :]
    hw_p = jnp.pad(bld['head_w'], ((0, 0), (0, 0), (0, NCP - NC)))
    hb_p = jnp.pad(bld['head_b'], ((0, 0), (0, NCP - NC)))[:, None, :]
    ln1 = jnp.stack([bld['ln1_g'], bld['ln1_b']], axis=1)
    ln2 = jnp.stack([bld['ln2_g'], bld['ln2_b']], axis=1)
    lnf = jnp.stack([bld['lnf_g'], bld['lnf_b']], axis=1)

    rw = jnp.tensordot(coeffs_t, params['router_w'], axes=(0, 0))
    rb = jnp.tensordot(coeffs_t, params['router_b'], axes=(0, 0))
    logits = x.reshape(B, FLAT) @ rw + rb
    dest, bexp = pl.pallas_call(
        _route_kern,
        in_specs=[pl.BlockSpec((B, E), lambda: (0, 0))],
        out_specs=[pl.BlockSpec((B, 1), lambda: (0, 0)),
                   pl.BlockSpec((NBLK + 1, 1), lambda: (0, 0))],
        out_shape=[jax.ShapeDtypeStruct((B, 1), jnp.int32),
                   jax.ShapeDtypeStruct((NBLK + 1, 1), jnp.int32)],
    )(logits)

    patches = x.reshape(B, 3, 8, P, 8, P).transpose(0, 2, 4, 1, 3, 5).reshape(B, NP, 48)
    prow = jnp.concatenate(
        [patches, jnp.zeros((B, NP, 1), f32), jnp.ones((B, NP, 1), f32),
         jnp.zeros((B, NP, CIN - 50), f32)], axis=-1)
    c0 = jnp.zeros((CIN,), f32).at[48].set(1.0)
    row0 = jnp.broadcast_to(c0, (B, 1, CIN))
    xp = jnp.concatenate([row0, prow, jnp.zeros((B, SEQP - SEQ, CIN), f32)],
                         axis=1).reshape(B, SEQP * CIN)
    src = pl.pallas_call(
        _invert_kern,
        in_specs=[pl.BlockSpec((1, B), lambda: (0, 0))],
        out_specs=pl.BlockSpec((SLOTS, 1), lambda: (0, 0)),
        out_shape=jax.ShapeDtypeStruct((SLOTS, 1), jnp.int32),
    )(dest.reshape(1, B)).reshape(SLOTS)
    xp_tab = jnp.concatenate([xp, jnp.zeros((8, DW), f32)], axis=0)
    xg = pl.kernel(
        _sc_gather_body,
        out_type=jax.ShapeDtypeStruct((SLOTS, DW), f32),
        mesh=plsc.VectorSubcoreMesh(core_axis_name="c", subcore_axis_name="s"),
        scratch_types=[pltpu.VMEM((GCH,), jnp.int32),
                       pltpu.VMEM((GCH, DW), f32),
                       pltpu.SemaphoreType.DMA],
    )(xp_tab, src).reshape(SLOTS * SEQP, CIN)

    grid_spec = pltpu.PrefetchScalarGridSpec(
        num_scalar_prefetch=1,
        grid=(NBLK,),
        in_specs=[
            pl.BlockSpec((BLK * SEQP, CIN), lambda i, be: (i, 0)),
            pl.BlockSpec((1, CIN, D), lambda i, be: (be[i], 0, 0)),
            pl.BlockSpec((1, SEQP, D), lambda i, be: (be[i], 0, 0)),
            pl.BlockSpec((1, 2, D), lambda i, be: (be[i], 0, 0)),
            pl.BlockSpec((1, D, 3 * D), lambda i, be: (be[i], 0, 0)),
            pl.BlockSpec((1, 1, 3 * D), lambda i, be: (be[i], 0, 0)),
            pl.BlockSpec((1, D, D), lambda i, be: (be[i], 0, 0)),
            pl.BlockSpec((1, 1, D), lambda i, be: (be[i], 0, 0)),
            pl.BlockSpec((1, 2, D), lambda i, be: (be[i], 0, 0)),
            pl.BlockSpec((1, D, 4 * D), lambda i, be: (be[i], 0, 0)),
            pl.BlockSpec((1, 1, 4 * D), lambda i, be: (be[i], 0, 0)),
            pl.BlockSpec((1, 4 * D, D), lambda i, be: (be[i], 0, 0)),
            pl.BlockSpec((1, 1, D), lambda i, be: (be[i], 0, 0)),
            pl.BlockSpec((1, 2, D), lambda i, be: (be[i], 0, 0)),
            pl.BlockSpec((1, D, NCP), lambda i, be: (be[i], 0, 0)),
            pl.BlockSpec((1, 1, NCP), lambda i, be: (be[i], 0, 0)),
        ],
        out_specs=pl.BlockSpec((BLK, NCP), lambda i, be: (i, 0)),
    )
    slot_out = pl.pallas_call(
        _expert_kern,
        grid_spec=grid_spec,
        out_shape=jax.ShapeDtypeStruct((SLOTS, NCP), f32),
    )(bexp.reshape(NBLK + 1), xg, pw_pad, pos_p, ln1, qkvw, qkvb, pj, pjb,
      ln2, w1, b1r, w2, b2r, lnf, hw_p, hb_p)

    out = pl.kernel(
        _sc_unscatter_body,
        out_type=jax.ShapeDtypeStruct((B, NCP), f32),
        mesh=plsc.VectorSubcoreMesh(core_axis_name="c", subcore_axis_name="s"),
        scratch_types=[pltpu.VMEM((UPW,), jnp.int32),
                       pltpu.VMEM((UPW, NCP), f32),
                       pltpu.SemaphoreType.DMA],
    )(slot_out, dest.reshape(B))
    return out[:, :NC]


# speedup vs baseline: 6.7525x; 1.0985x over previous
import jax
import jax.numpy as jnp
import numpy as np
from jax.experimental import pallas as pl
from jax.experimental.pallas import tpu as pltpu
from jax.experimental.pallas import tpu_sc as plsc

E, K, NC, D, P, NP, H, DH = 8, 3, 100, 192, 4, 64, 3, 64
B = 1024
FLAT = 3 * 32 * 32
SEQ = NP + 1
SEQP = 72
CIN = 64
NCP = 128
BLK = 32
NBLK = B // BLK + E
SLOTS = NBLK * BLK
DW = SEQP * CIN
SC_NC, SC_NS = 2, 16
NW = SC_NC * SC_NS
GPW = SLOTS // NW
GCH = 8
UPW = B // NW

_HI = jax.lax.Precision.HIGHEST
_DF = jax.lax.Precision.DEFAULT

_BLEND_NAMES = ['patch_w', 'patch_b', 'cls', 'pos', 'ln1_g', 'ln1_b', 'qkv_w',
                'qkv_b', 'proj_w', 'proj_b', 'ln2_g', 'ln2_b', 'mlp_w1',
                'mlp_b1', 'mlp_w2', 'mlp_b2', 'lnf_g', 'lnf_b', 'head_w',
                'head_b']


def _blend_kern(s_ref, *refs):
    n = len(refs) // 2
    s = s_ref[...]
    for i_ref, o_ref in zip(refs[:n], refs[n:]):
        o_ref[...] = jnp.dot(s, i_ref[...], precision=_DF)


def _route_kern(lg_ref, dest_ref, bexp_ref):
    logits = lg_ref[...]
    m = jnp.max(logits, axis=-1, keepdims=True)
    eiota = jax.lax.broadcasted_iota(jnp.int32, (B, E), 1)
    idx = jnp.min(jnp.where(logits == m, eiota, E), axis=-1, keepdims=True)
    oh = (idx == eiota).astype(jnp.float32)
    ii = jax.lax.broadcasted_iota(jnp.int32, (B, B), 0)
    jj = jax.lax.broadcasted_iota(jnp.int32, (B, B), 1)
    ltri = (jj < ii).astype(jnp.float32)
    rank_all = jnp.dot(ltri, oh, precision=_HI)
    rank = jnp.sum(rank_all * oh, axis=-1, keepdims=True)
    counts = jnp.sum(oh, axis=0, keepdims=True)
    padded = jnp.ceil(counts / BLK) * BLK
    er = jax.lax.broadcasted_iota(jnp.int32, (E, E), 0)
    ec = jax.lax.broadcasted_iota(jnp.int32, (E, E), 1)
    ustrict = (er < ec).astype(jnp.float32)
    offs = jnp.dot(padded, ustrict, precision=_HI)
    dest = jnp.sum(oh * offs, axis=-1, keepdims=True) + rank
    dest_ref[...] = dest.astype(jnp.int32)
    rowi = jax.lax.broadcasted_iota(jnp.int32, (NBLK + 1, E), 0)
    pstart = (rowi * BLK).astype(jnp.float32)
    within = (pstart >= offs) & (pstart < offs + padded)
    eid = jax.lax.broadcasted_iota(jnp.int32, (NBLK + 1, E), 1)
    base = jnp.sum(jnp.where(within, eid, 0), axis=-1, keepdims=True)
    nused = (jnp.sum(padded) * (1.0 / BLK)).astype(jnp.int32)
    bexp_ref[...] = base + jnp.where(rowi[:, :1] == NBLK, nused, 0)


def _gather_kern(dr_ref, xp_ref, xg_ref):
    b = pl.program_id(0)
    rows = jax.lax.broadcasted_iota(jnp.int32, (BLK, B), 0) + b * BLK
    oh = (rows == dr_ref[...]).astype(jnp.float32)
    xg_ref[...] = jnp.dot(oh, xp_ref[...], precision=_DF)


def _invert_kern(dr_ref, src_ref):
    rows = jax.lax.broadcasted_iota(jnp.int32, (SLOTS, B), 0)
    ohs = (rows == dr_ref[...]).astype(jnp.float32)
    ic = jax.lax.broadcasted_iota(jnp.int32, (B, 1), 0).astype(jnp.float32)
    valid = jnp.sum(ohs, axis=-1, keepdims=True)
    srcf = jnp.dot(ohs, ic, precision=_HI) + (1.0 - valid) * B
    src_ref[...] = srcf.astype(jnp.int32)


def _sc_gather_body(tab_ref, idx_ref, out_ref, idx_v, rows_v, sem):
    wid = jax.lax.axis_index("s") * SC_NC + jax.lax.axis_index("c")
    base = wid * GPW
    for c in range(GPW // GCH):
        off = base + c * GCH
        pltpu.sync_copy(idx_ref.at[pl.ds(off, GCH)], idx_v)
        pltpu.async_copy(tab_ref.at[idx_v], rows_v, sem).wait()
        pltpu.sync_copy(rows_v, out_ref.at[pl.ds(off, GCH)])


def _sc_unscatter_body(tab_ref, idx_ref, out_ref, idx_v, rows_v, sem):
    wid = jax.lax.axis_index("s") * SC_NC + jax.lax.axis_index("c")
    base = wid * UPW
    pltpu.sync_copy(idx_ref.at[pl.ds(base, UPW)], idx_v)
    pltpu.async_copy(tab_ref.at[idx_v], rows_v, sem).wait()
    pltpu.sync_copy(rows_v, out_ref.at[pl.ds(base, UPW)])


def _lnk(x, g, b):
    m = x.mean(-1, keepdims=True)
    v = ((x - m) ** 2).mean(-1, keepdims=True)
    return (x - m) / jnp.sqrt(v + 1e-6) * g + b


def _expert_kern(bexp, xg_ref, pw_ref, pos_ref, ln1_ref, qkvw_ref, qkvb_ref,
                 pj_ref, pjb_ref, ln2_ref, w1_ref, b1_ref, w2_ref, b2_ref,
                 lnf_ref, hw_ref, hb_ref, out_ref):
    nused = bexp[NBLK]

    @pl.when(pl.program_id(0) < nused)
    def _body():
        _expert_compute(bexp, xg_ref, pw_ref, pos_ref, ln1_ref, qkvw_ref,
                        qkvb_ref, pj_ref, pjb_ref, ln2_ref, w1_ref, b1_ref,
                        w2_ref, b2_ref, lnf_ref, hw_ref, hb_ref, out_ref)


def _expert_compute(bexp, xg_ref, pw_ref, pos_ref, ln1_ref, qkvw_ref, qkvb_ref,
                    pj_ref, pjb_ref, ln2_ref, w1_ref, b1_ref, w2_ref, b2_ref,
                    lnf_ref, hw_ref, hb_ref, out_ref):
    xg = xg_ref[...]
    t2 = jnp.dot(xg, pw_ref[0], precision=_DF)
    t = t2.reshape(BLK, SEQP, D) + pos_ref[0]
    h = _lnk(t, ln1_ref[0, 0:1, :], ln1_ref[0, 1:2, :])
    h2 = h.reshape(BLK * SEQP, D)
    qkv = jnp.dot(h2, qkvw_ref[0], precision=_DF) + qkvb_ref[0]
    colmask = (jax.lax.broadcasted_iota(jnp.int32, (1, 1, SEQP), 2) < SEQ)
    attn_out = jnp.zeros((BLK * SEQP, D), dtype=jnp.float32)
    for hh in range(H):
        q3 = qkv[:, hh * DH:(hh + 1) * DH].reshape(BLK, SEQP, DH)
        k3 = qkv[:, D + hh * DH:D + (hh + 1) * DH].reshape(BLK, SEQP, DH)
        v3 = qkv[:, 2 * D + hh * DH:2 * D + (hh + 1) * DH].reshape(BLK, SEQP, DH)
        sc = jax.lax.dot_general(q3, k3, (((2,), (2,)), ((0,), (0,))),
                                 precision=_DF) * (1.0 / np.sqrt(DH))
        sc = jnp.where(colmask, sc, -1e9)
        att = jax.nn.softmax(sc, axis=-1)
        o3 = jax.lax.dot_general(att, v3, (((2,), (1,)), ((0,), (0,))),
                                 precision=_DF)
        attn_out = attn_out + jnp.dot(o3.reshape(BLK * SEQP, DH), pj_ref[0, hh],
                                      precision=_DF)
    t = t + attn_out.reshape(BLK, SEQP, D) + pjb_ref[0]
    h = _lnk(t, ln2_ref[0, 0:1, :], ln2_ref[0, 1:2, :])
    h2 = h.reshape(BLK * SEQP, D)
    mh = jax.nn.gelu(jnp.dot(h2, w1_ref[0], precision=_DF) + b1_ref[0])
    m2 = jnp.dot(mh, w2_ref[0], precision=_DF) + b2_ref[0]
    t = t + m2.reshape(BLK, SEQP, D)
    f = t[:, 0, :]
    f = _lnk(f, lnf_ref[0, 0:1, :], lnf_ref[0, 1:2, :])
    out_ref[...] = jnp.dot(f, hw_ref[0], precision=_DF) + hb_ref[0]


def _unscatter_kern(dest_ref, so_ref, out_ref):
    cols = jax.lax.broadcasted_iota(jnp.int32, (B, SLOTS), 1)
    oh = (cols == dest_ref[...]).astype(jnp.float32)
    out_ref[...] = jnp.dot(oh, so_ref[...], precision=_HI)


def kernel(input, coeffs_t, params):
    x = input.astype(jnp.float32)
    f32 = jnp.float32

    ins, shapes = [], {}
    for name in _BLEND_NAMES:
        v = params[name]
        shapes[name] = (E,) + v.shape[2:]
        ins.append(v.reshape(E * K, -1))
    sel = jnp.kron(jnp.eye(E, dtype=f32), coeffs_t.astype(f32)[None, :])
    outs = []
    for lo, hi in ((0, 13), (13, len(ins))):
        grp = ins[lo:hi]
        outs += pl.pallas_call(
            _blend_kern,
            in_specs=([pl.BlockSpec((E, E * K), lambda: (0, 0))] +
                      [pl.BlockSpec(a.shape, lambda: (0, 0)) for a in grp]),
            out_specs=[pl.BlockSpec((E, a.shape[1]), lambda: (0, 0)) for a in grp],
            out_shape=[jax.ShapeDtypeStruct((E, a.shape[1]), f32) for a in grp],
        )(sel, *grp)
    bld = {name: o.reshape(shapes[name]) for name, o in zip(_BLEND_NAMES, outs)}

    pw_pad = jnp.concatenate(
        [bld['patch_w'], bld['cls'], bld['patch_b'][:, None, :],
         jnp.zeros((E, CIN - 50, D), f32)], axis=1)
    pos_p = jnp.pad(bld['pos'], ((0, 0), (0, SEQP - SEQ), (0, 0)))
    qkvw = bld['qkv_w']
    qkvb = bld['qkv_b'][:, None, :]
    pj = bld['proj_w'].reshape(E, H, DH, D)
    pjb = bld['proj_b'][:, None, :]
    w1, b1r = bld['mlp_w1'], bld['mlp_b1'][:, None, :]
    w2, b2r = bld['mlp_w2'], bld['mlp_b2'][:, None, :]
    hw_p = jnp.pad(bld['head_w'], ((0, 0), (0, 0), (0, NCP - NC)))
    hb_p = jnp.pad(bld['head_b'], ((0, 0), (0, NCP - NC)))[:, None, :]
    ln1 = jnp.stack([bld['ln1_g'], bld['ln1_b']], axis=1)
    ln2 = jnp.stack([bld['ln2_g'], bld['ln2_b']], axis=1)
    lnf = jnp.stack([bld['lnf_g'], bld['lnf_b']], axis=1)

    rw = jnp.tensordot(coeffs_t, params['router_w'], axes=(0, 0))
    rb = jnp.tensordot(coeffs_t, params['router_b'], axes=(0, 0))
    logits = x.reshape(B, FLAT) @ rw + rb
    dest, bexp = pl.pallas_call(
        _route_kern,
        in_specs=[pl.BlockSpec((B, E), lambda: (0, 0))],
        out_specs=[pl.BlockSpec((B, 1), lambda: (0, 0)),
                   pl.BlockSpec((NBLK + 1, 1), lambda: (0, 0))],
        out_shape=[jax.ShapeDtypeStruct((B, 1), jnp.int32),
                   jax.ShapeDtypeStruct((NBLK + 1, 1), jnp.int32)],
    )(logits)

    patches = x.reshape(B, 3, 8, P, 8, P).transpose(0, 2, 4, 1, 3, 5).reshape(B, NP, 48)
    prow = jnp.concatenate(
        [patches, jnp.zeros((B, NP, 1), f32), jnp.ones((B, NP, 1), f32),
         jnp.zeros((B, NP, CIN - 50), f32)], axis=-1)
    c0 = jnp.zeros((CIN,), f32).at[48].set(1.0)
    row0 = jnp.broadcast_to(c0, (B, 1, CIN))
    xp = jnp.concatenate([row0, prow, jnp.zeros((B, SEQP - SEQ, CIN), f32)],
                         axis=1).reshape(B, SEQP * CIN)
    src = pl.pallas_call(
        _invert_kern,
        in_specs=[pl.BlockSpec((1, B), lambda: (0, 0))],
        out_specs=pl.BlockSpec((SLOTS, 1), lambda: (0, 0)),
        out_shape=jax.ShapeDtypeStruct((SLOTS, 1), jnp.int32),
    )(dest.reshape(1, B)).reshape(SLOTS)
    xp_tab = jnp.concatenate([xp, jnp.zeros((8, DW), f32)], axis=0)
    xg = pl.kernel(
        _sc_gather_body,
        out_type=jax.ShapeDtypeStruct((SLOTS, DW), f32),
        mesh=plsc.VectorSubcoreMesh(core_axis_name="c", subcore_axis_name="s"),
        scratch_types=[pltpu.VMEM((GCH,), jnp.int32),
                       pltpu.VMEM((GCH, DW), f32),
                       pltpu.SemaphoreType.DMA],
    )(xp_tab, src).reshape(SLOTS * SEQP, CIN)

    grid_spec = pltpu.PrefetchScalarGridSpec(
        num_scalar_prefetch=1,
        grid=(NBLK,),
        in_specs=[
            pl.BlockSpec((BLK * SEQP, CIN), lambda i, be: (i, 0)),
            pl.BlockSpec((1, CIN, D), lambda i, be: (be[i], 0, 0)),
            pl.BlockSpec((1, SEQP, D), lambda i, be: (be[i], 0, 0)),
            pl.BlockSpec((1, 2, D), lambda i, be: (be[i], 0, 0)),
            pl.BlockSpec((1, D, 3 * D), lambda i, be: (be[i], 0, 0)),
            pl.BlockSpec((1, 1, 3 * D), lambda i, be: (be[i], 0, 0)),
            pl.BlockSpec((1, H, DH, D), lambda i, be: (be[i], 0, 0, 0)),
            pl.BlockSpec((1, 1, D), lambda i, be: (be[i], 0, 0)),
            pl.BlockSpec((1, 2, D), lambda i, be: (be[i], 0, 0)),
            pl.BlockSpec((1, D, 4 * D), lambda i, be: (be[i], 0, 0)),
            pl.BlockSpec((1, 1, 4 * D), lambda i, be: (be[i], 0, 0)),
            pl.BlockSpec((1, 4 * D, D), lambda i, be: (be[i], 0, 0)),
            pl.BlockSpec((1, 1, D), lambda i, be: (be[i], 0, 0)),
            pl.BlockSpec((1, 2, D), lambda i, be: (be[i], 0, 0)),
            pl.BlockSpec((1, D, NCP), lambda i, be: (be[i], 0, 0)),
            pl.BlockSpec((1, 1, NCP), lambda i, be: (be[i], 0, 0)),
        ],
        out_specs=pl.BlockSpec((BLK, NCP), lambda i, be: (i, 0)),
    )
    slot_out = pl.pallas_call(
        _expert_kern,
        grid_spec=grid_spec,
        out_shape=jax.ShapeDtypeStruct((SLOTS, NCP), f32),
    )(bexp.reshape(NBLK + 1), xg, pw_pad, pos_p, ln1, qkvw, qkvb, pj, pjb,
      ln2, w1, b1r, w2, b2r, lnf, hw_p, hb_p)

    out = pl.kernel(
        _sc_unscatter_body,
        out_type=jax.ShapeDtypeStruct((B, NCP), f32),
        mesh=plsc.VectorSubcoreMesh(core_axis_name="c", subcore_axis_name="s"),
        scratch_types=[pltpu.VMEM((UPW,), jnp.int32),
                       pltpu.VMEM((UPW, NCP), f32),
                       pltpu.SemaphoreType.DMA],
    )(slot_out, dest.reshape(B))
    return out[:, :NC]


# speedup vs baseline: 6.7550x; 1.0004x over previous
import jax
import jax.numpy as jnp
import numpy as np
from jax.experimental import pallas as pl
from jax.experimental.pallas import tpu as pltpu
from jax.experimental.pallas import tpu_sc as plsc

E, K, NC, D, P, NP, H, DH = 8, 3, 100, 192, 4, 64, 3, 64
B = 1024
FLAT = 3 * 32 * 32
SEQ = NP + 1
SEQP = 72
CIN = 64
NCP = 128
BLK = 32
NBLK = B // BLK + E
SLOTS = NBLK * BLK
DW = SEQP * CIN
SC_NC, SC_NS = 2, 16
NW = SC_NC * SC_NS
GPW = SLOTS // NW
GCH = 8
UPW = B // NW

_HI = jax.lax.Precision.HIGHEST
_DF = jax.lax.Precision.DEFAULT

_BLEND_NAMES = ['patch_w', 'patch_b', 'cls', 'pos', 'ln1_g', 'ln1_b', 'qkv_w',
                'qkv_b', 'proj_w', 'proj_b', 'ln2_g', 'ln2_b', 'mlp_w1',
                'mlp_b1', 'mlp_w2', 'mlp_b2', 'lnf_g', 'lnf_b', 'head_w',
                'head_b']


def _blend_kern(s_ref, *refs):
    n = len(refs) // 2
    s = s_ref[...]
    for i_ref, o_ref in zip(refs[:n], refs[n:]):
        o_ref[...] = jnp.dot(s, i_ref[...], precision=_DF)


def _route_kern(lg_ref, dest_ref, bexp_ref):
    logits = lg_ref[...]
    m = jnp.max(logits, axis=-1, keepdims=True)
    eiota = jax.lax.broadcasted_iota(jnp.int32, (B, E), 1)
    idx = jnp.min(jnp.where(logits == m, eiota, E), axis=-1, keepdims=True)
    oh = (idx == eiota).astype(jnp.float32)
    ii = jax.lax.broadcasted_iota(jnp.int32, (B, B), 0)
    jj = jax.lax.broadcasted_iota(jnp.int32, (B, B), 1)
    ltri = (jj < ii).astype(jnp.float32)
    rank_all = jnp.dot(ltri, oh, precision=_HI)
    rank = jnp.sum(rank_all * oh, axis=-1, keepdims=True)
    counts = jnp.sum(oh, axis=0, keepdims=True)
    padded = jnp.ceil(counts / BLK) * BLK
    er = jax.lax.broadcasted_iota(jnp.int32, (E, E), 0)
    ec = jax.lax.broadcasted_iota(jnp.int32, (E, E), 1)
    ustrict = (er < ec).astype(jnp.float32)
    offs = jnp.dot(padded, ustrict, precision=_HI)
    dest = jnp.sum(oh * offs, axis=-1, keepdims=True) + rank
    dest_ref[...] = dest.astype(jnp.int32)
    rowi = jax.lax.broadcasted_iota(jnp.int32, (NBLK + 1, E), 0)
    pstart = (rowi * BLK).astype(jnp.float32)
    within = (pstart >= offs) & (pstart < offs + padded)
    eid = jax.lax.broadcasted_iota(jnp.int32, (NBLK + 1, E), 1)
    base = jnp.sum(jnp.where(within, eid, 0), axis=-1, keepdims=True)
    nused = (jnp.sum(padded) * (1.0 / BLK)).astype(jnp.int32)
    bexp_ref[...] = base + jnp.where(rowi[:, :1] == NBLK, nused, 0)


def _invert_kern(dr_ref, src_ref):
    rows = jax.lax.broadcasted_iota(jnp.int32, (SLOTS, B), 0)
    ohs = (rows == dr_ref[...]).astype(jnp.float32)
    ic = jax.lax.broadcasted_iota(jnp.int32, (B, 1), 0).astype(jnp.float32)
    valid = jnp.sum(ohs, axis=-1, keepdims=True)
    srcf = jnp.dot(ohs, ic, precision=_HI) + (1.0 - valid) * B
    src_ref[...] = srcf.astype(jnp.int32)


def _sc_gather_body(tab_ref, idx_ref, out_ref, idx_v, rows_v, sem):
    wid = jax.lax.axis_index("s") * SC_NC + jax.lax.axis_index("c")
    base = wid * GPW
    for c in range(GPW // GCH):
        off = base + c * GCH
        pltpu.sync_copy(idx_ref.at[pl.ds(off, GCH)], idx_v)
        pltpu.async_copy(tab_ref.at[idx_v], rows_v, sem).wait()
        pltpu.sync_copy(rows_v, out_ref.at[pl.ds(off, GCH)])


def _sc_unscatter_body(tab_ref, idx_ref, out_ref, idx_v, rows_v, sem):
    wid = jax.lax.axis_index("s") * SC_NC + jax.lax.axis_index("c")
    base = wid * UPW
    pltpu.sync_copy(idx_ref.at[pl.ds(base, UPW)], idx_v)
    pltpu.async_copy(tab_ref.at[idx_v], rows_v, sem).wait()
    pltpu.sync_copy(rows_v, out_ref.at[pl.ds(base, UPW)])


def _lnk(x, g, b):
    m = x.mean(-1, keepdims=True)
    v = ((x - m) ** 2).mean(-1, keepdims=True)
    return (x - m) / jnp.sqrt(v + 1e-6) * g + b


def _expert_kern(bexp, xg_ref, pw_ref, pos_ref, ln1_ref, qkvw_ref, qkvb_ref,
                 pj_ref, pjb_ref, ln2_ref, w1_ref, b1_ref, w2_ref, b2_ref,
                 lnf_ref, hw_ref, hb_ref, out_ref):
    nused = bexp[NBLK]

    @pl.when(pl.program_id(0) < nused)
    def _body():
        _expert_compute(bexp, xg_ref, pw_ref, pos_ref, ln1_ref, qkvw_ref,
                        qkvb_ref, pj_ref, pjb_ref, ln2_ref, w1_ref, b1_ref,
                        w2_ref, b2_ref, lnf_ref, hw_ref, hb_ref, out_ref)


def _expert_compute(bexp, xg_ref, pw_ref, pos_ref, ln1_ref, qkvw_ref, qkvb_ref,
                    pj_ref, pjb_ref, ln2_ref, w1_ref, b1_ref, w2_ref, b2_ref,
                    lnf_ref, hw_ref, hb_ref, out_ref):
    xg = xg_ref[...]
    t2 = jnp.dot(xg, pw_ref[0], precision=_DF)
    t = t2.reshape(BLK, SEQP, D) + pos_ref[0]
    h = _lnk(t, ln1_ref[0, 0:1, :], ln1_ref[0, 1:2, :])
    h2 = h.reshape(BLK * SEQP, D)
    qkv = jnp.dot(h2, qkvw_ref[0], precision=_DF) + qkvb_ref[0]
    colmask = (jax.lax.broadcasted_iota(jnp.int32, (1, 1, SEQP), 2) < SEQ)
    attn_out = jnp.zeros((BLK * SEQP, D), dtype=jnp.float32)
    for hh in range(H):
        q3 = qkv[:, hh * DH:(hh + 1) * DH].reshape(BLK, SEQP, DH)
        k3 = qkv[:, D + hh * DH:D + (hh + 1) * DH].reshape(BLK, SEQP, DH)
        v3 = qkv[:, 2 * D + hh * DH:2 * D + (hh + 1) * DH].reshape(BLK, SEQP, DH)
        sc = jax.lax.dot_general(q3, k3, (((2,), (2,)), ((0,), (0,))),
                                 precision=_DF) * (1.0 / np.sqrt(DH))
        sc = jnp.where(colmask, sc, -1e9)
        att = jax.nn.softmax(sc, axis=-1)
        o3 = jax.lax.dot_general(att, v3, (((2,), (1,)), ((0,), (0,))),
                                 precision=_DF)
        attn_out = attn_out + jnp.dot(o3.reshape(BLK * SEQP, DH), pj_ref[0, hh],
                                      precision=_DF)
    t = t + attn_out.reshape(BLK, SEQP, D) + pjb_ref[0]
    h = _lnk(t, ln2_ref[0, 0:1, :], ln2_ref[0, 1:2, :])
    h2 = h.reshape(BLK * SEQP, D)
    mh = jax.nn.gelu(jnp.dot(h2, w1_ref[0], precision=_DF) + b1_ref[0])
    m2 = jnp.dot(mh, w2_ref[0], precision=_DF) + b2_ref[0]
    t = t + m2.reshape(BLK, SEQP, D)
    f = t[:, 0, :]
    f = _lnk(f, lnf_ref[0, 0:1, :], lnf_ref[0, 1:2, :])
    out_ref[...] = jnp.dot(f, hw_ref[0], precision=_DF) + hb_ref[0]


def kernel(input, coeffs_t, params):
    x = input.astype(jnp.float32)
    f32 = jnp.float32

    ins, shapes = [], {}
    for name in _BLEND_NAMES:
        v = params[name]
        shapes[name] = (E,) + v.shape[2:]
        ins.append(v.reshape(E * K, -1))
    sel = jnp.kron(jnp.eye(E, dtype=f32), coeffs_t.astype(f32)[None, :])
    outs = []
    for lo, hi in ((0, 13), (13, len(ins))):
        grp = ins[lo:hi]
        outs += pl.pallas_call(
            _blend_kern,
            in_specs=([pl.BlockSpec((E, E * K), lambda: (0, 0))] +
                      [pl.BlockSpec(a.shape, lambda: (0, 0)) for a in grp]),
            out_specs=[pl.BlockSpec((E, a.shape[1]), lambda: (0, 0)) for a in grp],
            out_shape=[jax.ShapeDtypeStruct((E, a.shape[1]), f32) for a in grp],
        )(sel, *grp)
    bld = {name: o.reshape(shapes[name]) for name, o in zip(_BLEND_NAMES, outs)}

    pw_pad = jnp.concatenate(
        [bld['patch_w'], bld['cls'], bld['patch_b'][:, None, :],
         jnp.zeros((E, CIN - 50, D), f32)], axis=1)
    pos_p = jnp.pad(bld['pos'], ((0, 0), (0, SEQP - SEQ), (0, 0)))
    qkvw = bld['qkv_w']
    qkvb = bld['qkv_b'][:, None, :]
    pj = bld['proj_w'].reshape(E, H, DH, D)
    pjb = bld['proj_b'][:, None, :]
    w1, b1r = bld['mlp_w1'], bld['mlp_b1'][:, None, :]
    w2, b2r = bld['mlp_w2'], bld['mlp_b2'][:, None, :]
    hw_p = jnp.pad(bld['head_w'], ((0, 0), (0, 0), (0, NCP - NC)))
    hb_p = jnp.pad(bld['head_b'], ((0, 0), (0, NCP - NC)))[:, None, :]
    ln1 = jnp.stack([bld['ln1_g'], bld['ln1_b']], axis=1)
    ln2 = jnp.stack([bld['ln2_g'], bld['ln2_b']], axis=1)
    lnf = jnp.stack([bld['lnf_g'], bld['lnf_b']], axis=1)

    rw = jnp.tensordot(coeffs_t, params['router_w'], axes=(0, 0))
    rb = jnp.tensordot(coeffs_t, params['router_b'], axes=(0, 0))
    logits = x.reshape(B, FLAT) @ rw + rb
    dest, bexp = pl.pallas_call(
        _route_kern,
        in_specs=[pl.BlockSpec((B, E), lambda: (0, 0))],
        out_specs=[pl.BlockSpec((B, 1), lambda: (0, 0)),
                   pl.BlockSpec((NBLK + 1, 1), lambda: (0, 0))],
        out_shape=[jax.ShapeDtypeStruct((B, 1), jnp.int32),
                   jax.ShapeDtypeStruct((NBLK + 1, 1), jnp.int32)],
    )(logits)

    patches = x.reshape(B, 3, 8, P, 8, P).transpose(0, 2, 4, 1, 3, 5).reshape(B, NP, 48)
    prow = jnp.concatenate(
        [patches, jnp.zeros((B, NP, 1), f32), jnp.ones((B, NP, 1), f32),
         jnp.zeros((B, NP, CIN - 50), f32)], axis=-1)
    c0 = jnp.zeros((CIN,), f32).at[48].set(1.0)
    row0 = jnp.broadcast_to(c0, (B, 1, CIN))
    xp = jnp.concatenate([row0, prow, jnp.zeros((B, SEQP - SEQ, CIN), f32)],
                         axis=1).reshape(B, SEQP * CIN)
    src = pl.pallas_call(
        _invert_kern,
        in_specs=[pl.BlockSpec((1, B), lambda: (0, 0))],
        out_specs=pl.BlockSpec((SLOTS, 1), lambda: (0, 0)),
        out_shape=jax.ShapeDtypeStruct((SLOTS, 1), jnp.int32),
    )(dest.reshape(1, B)).reshape(SLOTS)
    xp_tab = jnp.concatenate([xp, jnp.zeros((8, DW), f32)], axis=0)
    xg = pl.kernel(
        _sc_gather_body,
        out_type=jax.ShapeDtypeStruct((SLOTS, DW), f32),
        mesh=plsc.VectorSubcoreMesh(core_axis_name="c", subcore_axis_name="s"),
        scratch_types=[pltpu.VMEM((GCH,), jnp.int32),
                       pltpu.VMEM((GCH, DW), f32),
                       pltpu.SemaphoreType.DMA],
    )(xp_tab, src).reshape(SLOTS * SEQP, CIN)

    grid_spec = pltpu.PrefetchScalarGridSpec(
        num_scalar_prefetch=1,
        grid=(NBLK,),
        in_specs=[
            pl.BlockSpec((BLK * SEQP, CIN), lambda i, be: (i, 0)),
            pl.BlockSpec((1, CIN, D), lambda i, be: (be[i], 0, 0)),
            pl.BlockSpec((1, SEQP, D), lambda i, be: (be[i], 0, 0)),
            pl.BlockSpec((1, 2, D), lambda i, be: (be[i], 0, 0)),
            pl.BlockSpec((1, D, 3 * D), lambda i, be: (be[i], 0, 0)),
            pl.BlockSpec((1, 1, 3 * D), lambda i, be: (be[i], 0, 0)),
            pl.BlockSpec((1, H, DH, D), lambda i, be: (be[i], 0, 0, 0)),
            pl.BlockSpec((1, 1, D), lambda i, be: (be[i], 0, 0)),
            pl.BlockSpec((1, 2, D), lambda i, be: (be[i], 0, 0)),
            pl.BlockSpec((1, D, 4 * D), lambda i, be: (be[i], 0, 0)),
            pl.BlockSpec((1, 1, 4 * D), lambda i, be: (be[i], 0, 0)),
            pl.BlockSpec((1, 4 * D, D), lambda i, be: (be[i], 0, 0)),
            pl.BlockSpec((1, 1, D), lambda i, be: (be[i], 0, 0)),
            pl.BlockSpec((1, 2, D), lambda i, be: (be[i], 0, 0)),
            pl.BlockSpec((1, D, NCP), lambda i, be: (be[i], 0, 0)),
            pl.BlockSpec((1, 1, NCP), lambda i, be: (be[i], 0, 0)),
        ],
        out_specs=pl.BlockSpec((BLK, NCP), lambda i, be: (i, 0)),
    )
    slot_out = pl.pallas_call(
        _expert_kern,
        grid_spec=grid_spec,
        out_shape=jax.ShapeDtypeStruct((SLOTS, NCP), f32),
    )(bexp.reshape(NBLK + 1), xg, pw_pad, pos_p, ln1, qkvw, qkvb, pj, pjb,
      ln2, w1, b1r, w2, b2r, lnf, hw_p, hb_p)

    out = pl.kernel(
        _sc_unscatter_body,
        out_type=jax.ShapeDtypeStruct((B, NCP), f32),
        mesh=plsc.VectorSubcoreMesh(core_axis_name="c", subcore_axis_name="s"),
        scratch_types=[pltpu.VMEM((UPW,), jnp.int32),
                       pltpu.VMEM((UPW, NCP), f32),
                       pltpu.SemaphoreType.DMA],
    )(slot_out, dest.reshape(B))
    return out[:, :NC]
